# baseline (speedup 1.0000x reference)
import functools

import jax
import jax.numpy as jnp
from jax import lax
from jax.experimental import pallas as pl
from jax.experimental.pallas import tpu as pltpu
from jax.experimental.pallas import tpu_sc as plsc

_PRETRAINED_SIZE = 100000
_TRAINABLE_SIZE = 1000
_EMBED_DIM = 128
_BATCH = 4096
_HIST = 200

_N = _BATCH * _HIST

_info = plsc.get_sparse_core_info()
_NC = _info.num_cores
_NS = _info.num_subcores
_NW = _NC * _NS
_B_PER_W = _N // _NW
_CHUNK = 200
_NCHUNK = _B_PER_W // _CHUNK

_mesh = plsc.VectorSubcoreMesh(core_axis_name="c", subcore_axis_name="s")


@functools.partial(
    pl.kernel,
    mesh=_mesh,
    out_type=jax.ShapeDtypeStruct((_N, _EMBED_DIM), jnp.float32),
    scratch_types=[
        pltpu.VMEM((_B_PER_W,), jnp.int32),
        pltpu.VMEM((2, _CHUNK, _EMBED_DIM), jnp.float32),
        pltpu.VMEM_SHARED((_NS, 2, _CHUNK, _EMBED_DIM), jnp.float32),
        pltpu.SemaphoreType.DMA,
        pltpu.SemaphoreType.DMA,
        pltpu.SemaphoreType.DMA,
        pltpu.SemaphoreType.DMA,
        pltpu.SemaphoreType.DMA,
        pltpu.SemaphoreType.DMA,
    ],
)
def _gather_kernel(table_hbm, idx_hbm, out_hbm, idx_v, rows_v, rows_s,
                   sg0, sg1, sx0, sx1, sw0, sw1):
    sid = lax.axis_index("s")
    wid = sid * _NC + lax.axis_index("c")
    base = wid * _B_PER_W
    sg = (sg0, sg1)
    sx = (sx0, sx1)
    sw = (sw0, sw1)

    pltpu.sync_copy(idx_hbm.at[pl.ds(base, _B_PER_W)], idx_v)

    def fire_gather(i, b):
        pltpu.async_copy(
            table_hbm.at[idx_v.at[pl.ds(i * _CHUNK, _CHUNK)]],
            rows_v.at[b], sg[b])

    def wait_gather(b):
        pltpu.make_async_copy(
            table_hbm.at[pl.ds(0, _CHUNK)], rows_v.at[b], sg[b]).wait()

    def fire_stage(b):
        pltpu.async_copy(rows_v.at[b], rows_s.at[sid, b], sx[b])

    def wait_stage(b):
        pltpu.make_async_copy(
            rows_v.at[b], rows_s.at[sid, b], sx[b]).wait()

    def fire_write(i, b):
        pltpu.async_copy(
            rows_s.at[sid, b],
            out_hbm.at[pl.ds(base + i * _CHUNK, _CHUNK)], sw[b])

    def wait_write(b):
        pltpu.make_async_copy(
            rows_s.at[sid, b], out_hbm.at[pl.ds(0, _CHUNK)], sw[b]).wait()

    fire_gather(0, 0)
    fire_gather(1, 1)

    def body(j, carry):
        for b in range(2):
            i = 2 * j + b
            wait_gather(b)

            @pl.when(i >= 2)
            def _():
                wait_write(b)

            fire_stage(b)
            wait_stage(b)
            fire_write(i, b)

            @pl.when(i + 2 < _NCHUNK)
            def _():
                fire_gather(i + 2, b)

        return carry

    lax.fori_loop(0, _NCHUNK // 2, body, 0)
    wait_write(0)
    wait_write(1)


def kernel(idx, pretrained_w, trainable_w):
    table = jnp.concatenate([pretrained_w, trainable_w], axis=0)
    flat_idx = idx.reshape(-1).astype(jnp.int32)
    out = _gather_kernel(table, flat_idx)
    return out.reshape(_BATCH, _HIST, _EMBED_DIM)


# speedup vs baseline: 1.0020x; 1.0020x over previous
import functools

import jax
import jax.numpy as jnp
from jax import lax
from jax.experimental import pallas as pl
from jax.experimental.pallas import tpu as pltpu
from jax.experimental.pallas import tpu_sc as plsc

_PRETRAINED_SIZE = 100000
_TRAINABLE_SIZE = 1000
_EMBED_DIM = 128
_BATCH = 4096
_HIST = 200

_N = _BATCH * _HIST

_info = plsc.get_sparse_core_info()
_NC = _info.num_cores
_NS = _info.num_subcores
_NW = _NC * _NS
_B_PER_W = _N // _NW
_CHUNK = 128
_NCHUNK = _B_PER_W // _CHUNK

_mesh = plsc.VectorSubcoreMesh(core_axis_name="c", subcore_axis_name="s")


@functools.partial(
    pl.kernel,
    mesh=_mesh,
    out_type=jax.ShapeDtypeStruct((_N, _EMBED_DIM), jnp.float32),
    scratch_types=[
        pltpu.VMEM((_B_PER_W,), jnp.int32),
        pltpu.VMEM((4, _CHUNK, _EMBED_DIM), jnp.float32),
        pltpu.VMEM_SHARED((_NS, 2, _CHUNK, _EMBED_DIM), jnp.float32),
        pltpu.SemaphoreType.DMA,
        pltpu.SemaphoreType.DMA,
        pltpu.SemaphoreType.DMA,
        pltpu.SemaphoreType.DMA,
        pltpu.SemaphoreType.DMA,
        pltpu.SemaphoreType.DMA,
        pltpu.SemaphoreType.DMA,
        pltpu.SemaphoreType.DMA,
    ],
)
def _gather_kernel(table_hbm, idx_hbm, out_hbm, idx_v, rows_v, rows_s,
                   sg0, sg1, sg2, sg3, sx, swd, sws0, sws1):
    sid = lax.axis_index("s")
    wid = sid * _NC + lax.axis_index("c")
    base = wid * _B_PER_W
    sg = (sg0, sg1, sg2, sg3)
    sws = (sws0, sws1)

    pltpu.sync_copy(idx_hbm.at[pl.ds(base, _B_PER_W)], idx_v)

    def fire_gather(i, b):
        pltpu.async_copy(
            table_hbm.at[idx_v.at[pl.ds(i * _CHUNK, _CHUNK)]],
            rows_v.at[b], sg[b])

    def wait_gather(b):
        pltpu.make_async_copy(
            table_hbm.at[pl.ds(0, _CHUNK)], rows_v.at[b], sg[b]).wait()

    def fire_direct(i, b):
        pltpu.async_copy(
            rows_v.at[b], out_hbm.at[pl.ds(base + i * _CHUNK, _CHUNK)],
            swd)

    def wait_direct(b):
        pltpu.make_async_copy(
            rows_v.at[b], out_hbm.at[pl.ds(0, _CHUNK)], swd).wait()

    def fire_stage(b, s):
        pltpu.async_copy(rows_v.at[b], rows_s.at[sid, s], sx)

    def wait_stage(b, s):
        pltpu.make_async_copy(rows_v.at[b], rows_s.at[sid, s], sx).wait()

    def fire_ws(i, s):
        pltpu.async_copy(
            rows_s.at[sid, s],
            out_hbm.at[pl.ds(base + i * _CHUNK, _CHUNK)], sws[s])

    def wait_ws(s):
        pltpu.make_async_copy(
            rows_s.at[sid, s], out_hbm.at[pl.ds(0, _CHUNK)],
            sws[s]).wait()

    for b in range(4):
        fire_gather(b, b)

    slot_for_k = {1: 0, 2: 1, 3: 0, 5: 1, 6: 0, 7: 1}

    def body(j, carry):
        for k in range(8):
            c = 8 * j + k
            b = k % 4
            wait_gather(b)
            if k % 4 == 0:
                fire_direct(c, b)
                wait_direct(b)
            else:
                s = slot_for_k[k]
                if k in (1, 2):
                    @pl.when(j > 0)
                    def _():
                        wait_ws(s)
                else:
                    wait_ws(s)
                fire_stage(b, s)
                wait_stage(b, s)
                fire_ws(c, s)

            @pl.when(c + 4 < _NCHUNK)
            def _():
                fire_gather(c + 4, b)

        return carry

    lax.fori_loop(0, _NCHUNK // 8, body, 0)
    wait_ws(0)
    wait_ws(1)


def kernel(idx, pretrained_w, trainable_w):
    table = jnp.concatenate([pretrained_w, trainable_w], axis=0)
    flat_idx = idx.reshape(-1).astype(jnp.int32)
    out = _gather_kernel(table, flat_idx)
    return out.reshape(_BATCH, _HIST, _EMBED_DIM)
